# baseline (speedup 1.0000x reference)
import functools

import jax
import jax.numpy as jnp
from jax import lax
from jax.experimental import pallas as pl
from jax.experimental.pallas import tpu as pltpu
from jax.experimental.pallas import tpu_sc as plsc

N = 10000
D = 128
E = 320000

NC = 2
NS = 16
NW = NC * NS

CH = 64
CPW = 158
EPW = CPW * CH
E_PAD = NW * EPW
NACC = 10016
RPT = NACC // NS


def _make_agg(do_deg):
    mesh = plsc.VectorSubcoreMesh(core_axis_name="c", subcore_axis_name="s")
    out_type = [jax.ShapeDtypeStruct((NC, NACC, D), jnp.float32)]
    scratch = [
        pltpu.VMEM((CPW, CH), jnp.int32),
        pltpu.VMEM((CPW, CH), jnp.int32),
        pltpu.VMEM((CH, D), jnp.float32),
        pltpu.VMEM((CH, D), jnp.float32),
        pltpu.VMEM_SHARED((NACC, D), jnp.float32),
        pltpu.SemaphoreType.DMA,
        pltpu.SemaphoreType.DMA,
        pltpu.SemaphoreType.DMA,
        pltpu.SemaphoreType.DMA,
    ]
    if do_deg:
        out_type.append(jax.ShapeDtypeStruct((NC, NACC, 16), jnp.float32))
        scratch += [
            pltpu.VMEM((CH, 16), jnp.float32),
            pltpu.VMEM((CH, 16), jnp.float32),
            pltpu.VMEM_SHARED((NACC, 16), jnp.float32),
        ]

    @functools.partial(
        pl.kernel, mesh=mesh, out_type=out_type, scratch_types=scratch,
        compiler_params=pltpu.CompilerParams(use_tc_tiling_on_sc=False))
    def agg(*refs):
        if do_deg:
            (x_hbm, src_hbm, dst_hbm,
             agg_out, deg_out, src_v, dst_v, rows0, rows1, acc_sh,
             sem0, sem1, ssem0, ssem1, ones_v, zd_v, deg_sh) = refs
        else:
            (x_hbm, src_hbm, dst_hbm,
             agg_out, src_v, dst_v, rows0, rows1, acc_sh,
             sem0, sem1, ssem0, ssem1) = refs
        c = lax.axis_index("c")
        s = lax.axis_index("s")
        wid = c * NS + s

        z16 = jnp.zeros((16,), jnp.float32)
        o16 = jnp.ones((16,), jnp.float32)

        def fill(i, carry):
            for j in range(D // 16):
                rows0[i, pl.ds(j * 16, 16)] = z16
            if do_deg:
                ones_v[i, :] = o16
                zd_v[i, :] = z16
            return carry

        lax.fori_loop(0, CH, fill, 0)

        nfull = RPT // CH
        rem = RPT - nfull * CH
        for k in range(nfull):
            pltpu.sync_copy(rows0, acc_sh.at[pl.ds(s * RPT + k * CH, CH)])
        pltpu.sync_copy(rows0.at[pl.ds(0, rem)],
                        acc_sh.at[pl.ds(s * RPT + nfull * CH, rem)])
        if do_deg:
            for k in range(nfull):
                pltpu.sync_copy(zd_v, deg_sh.at[pl.ds(s * RPT + k * CH, CH)])
            pltpu.sync_copy(zd_v.at[pl.ds(0, rem)],
                            deg_sh.at[pl.ds(s * RPT + nfull * CH, rem)])
        pltpu.sync_copy(src_hbm.at[wid], src_v)
        pltpu.sync_copy(dst_hbm.at[wid], dst_v)
        plsc.subcore_barrier()

        def gath(buf, gsem, j):
            pltpu.async_copy(x_hbm.at[src_v.at[j]], buf, gsem)

        def gwait(buf, gsem, j):
            pltpu.make_async_copy(x_hbm.at[src_v.at[j]], buf, gsem).wait()

        def scat(buf, ssem, j):
            pltpu.async_copy(buf, acc_sh.at[dst_v.at[j]], ssem, add=True)
            if do_deg:
                pltpu.sync_copy(ones_v, deg_sh.at[dst_v.at[j]], add=True)

        def swait(buf, ssem, j):
            pltpu.make_async_copy(buf, acc_sh.at[dst_v.at[j]], ssem).wait()

        gath(rows0, sem0, 0)
        gath(rows1, sem1, 1)
        gwait(rows0, sem0, 0)
        scat(rows0, ssem0, 0)
        gwait(rows1, sem1, 1)
        scat(rows1, ssem1, 1)

        def body(g, carry):
            j0 = 2 * g + 2
            swait(rows0, ssem0, j0 - 2)
            gath(rows0, sem0, j0)
            swait(rows1, ssem1, j0 - 1)
            gath(rows1, sem1, j0 + 1)
            gwait(rows0, sem0, j0)
            scat(rows0, ssem0, j0)
            gwait(rows1, sem1, j0 + 1)
            scat(rows1, ssem1, j0 + 1)
            return carry

        lax.fori_loop(0, (CPW - 2) // 2, body, 0)
        swait(rows0, ssem0, CPW - 2)
        swait(rows1, ssem1, CPW - 1)
        plsc.subcore_barrier()

        ob = s * RPT
        pltpu.sync_copy(acc_sh.at[pl.ds(ob, RPT)], agg_out.at[c, pl.ds(ob, RPT)])
        if do_deg:
            pltpu.sync_copy(deg_sh.at[pl.ds(ob, RPT)], deg_out.at[c, pl.ds(ob, RPT)])

    return agg


_agg_with_deg = _make_agg(True)
_agg_no_deg = _make_agg(False)

_BN = 1000


def _dense_body(relu, agg_ref, deg_ref, x_ref, wl_ref, bl_ref, wr_ref, o_ref):
    agg = agg_ref[0] + agg_ref[1]
    deg = deg_ref[0, :, 0:1] + deg_ref[1, :, 0:1]
    mean = agg / jnp.maximum(deg, 1.0)
    y = (jnp.dot(mean, wl_ref[...], preferred_element_type=jnp.float32)
         + jnp.dot(x_ref[...], wr_ref[...], preferred_element_type=jnp.float32)
         + bl_ref[...])
    o_ref[...] = jnp.maximum(y, 0.0) if relu else y


def _dense(aggp, degp, x, WlT, bl, WrT, relu):
    return pl.pallas_call(
        functools.partial(_dense_body, relu),
        grid=(N // _BN,),
        in_specs=[
            pl.BlockSpec((NC, _BN, D), lambda i: (0, i, 0)),
            pl.BlockSpec((NC, _BN, 16), lambda i: (0, i, 0)),
            pl.BlockSpec((_BN, D), lambda i: (i, 0)),
            pl.BlockSpec((D, D), lambda i: (0, 0)),
            pl.BlockSpec((1, D), lambda i: (0, 0)),
            pl.BlockSpec((D, D), lambda i: (0, 0)),
        ],
        out_specs=pl.BlockSpec((_BN, D), lambda i: (i, 0)),
        out_shape=jax.ShapeDtypeStruct((N, D), jnp.float32),
    )(aggp, degp, x, WlT, bl, WrT)


def kernel(x, edge_index, Wl1, bl1, Wr1, Wl2, bl2, Wr2):
    src = edge_index[0]
    dst = edge_index[1]
    epw = E // NW
    pad_w = EPW - epw
    src_p = jnp.concatenate(
        [src.reshape(NW, epw), jnp.zeros((NW, pad_w), jnp.int32)], axis=1
    ).reshape(NW, CPW, CH)
    dummy = N + (jnp.arange(pad_w, dtype=jnp.int32) % (NACC - N))
    dst_p = jnp.concatenate(
        [dst.reshape(NW, epw), jnp.broadcast_to(dummy, (NW, pad_w))], axis=1
    ).reshape(NW, CPW, CH)
    agg1, deg = _agg_with_deg(x, src_p, dst_p)
    h1 = _dense(agg1, deg, x, Wl1.T, bl1.reshape(1, D), Wr1.T, relu=True)
    (agg2,) = _agg_no_deg(h1, src_p, dst_p)
    out = _dense(agg2, deg, h1, Wl2.T, bl2.reshape(1, D), Wr2.T, relu=False)
    return out


# speedup vs baseline: 1.1024x; 1.1024x over previous
import functools

import jax
import jax.numpy as jnp
from jax import lax
from jax.experimental import pallas as pl
from jax.experimental.pallas import tpu as pltpu
from jax.experimental.pallas import tpu_sc as plsc

N = 10000
D = 128
E = 320000

NC = 2
NS = 16
NW = NC * NS

CH = 64
CPW = 158
EPW = CPW * CH
E_PAD = NW * EPW
NACC = 10016
RPT = NACC // NS


def _make_agg(do_deg):
    mesh = plsc.VectorSubcoreMesh(core_axis_name="c", subcore_axis_name="s")
    out_type = [jax.ShapeDtypeStruct((NC, NACC, D), jnp.float32)]
    scratch = [
        pltpu.VMEM((CPW, CH), jnp.int32),
        pltpu.VMEM((CPW, CH), jnp.int32),
        pltpu.VMEM((CH, D), jnp.float32),
        pltpu.VMEM((CH, D), jnp.float32),
        pltpu.VMEM_SHARED((NACC, D), jnp.float32),
        pltpu.SemaphoreType.DMA,
        pltpu.SemaphoreType.DMA,
    ]
    if do_deg:
        out_type.append(jax.ShapeDtypeStruct((NC, NACC, 16), jnp.float32))
        scratch += [
            pltpu.VMEM((CH, 16), jnp.float32),
            pltpu.VMEM((CH, 16), jnp.float32),
            pltpu.VMEM_SHARED((NACC, 16), jnp.float32),
        ]

    @functools.partial(
        pl.kernel, mesh=mesh, out_type=out_type, scratch_types=scratch,
        compiler_params=pltpu.CompilerParams(use_tc_tiling_on_sc=False))
    def agg(*refs):
        if do_deg:
            (x_hbm, src_hbm, dst_hbm,
             agg_out, deg_out, src_v, dst_v, rows0, rows1, acc_sh, sem0, sem1,
             ones_v, zd_v, deg_sh) = refs
        else:
            (x_hbm, src_hbm, dst_hbm,
             agg_out, src_v, dst_v, rows0, rows1, acc_sh, sem0, sem1) = refs
        c = lax.axis_index("c")
        s = lax.axis_index("s")
        wid = c * NS + s

        z16 = jnp.zeros((16,), jnp.float32)
        o16 = jnp.ones((16,), jnp.float32)

        def fill(i, carry):
            for j in range(D // 16):
                rows0[i, pl.ds(j * 16, 16)] = z16
            if do_deg:
                ones_v[i, :] = o16
                zd_v[i, :] = z16
            return carry

        lax.fori_loop(0, CH, fill, 0)

        nfull = RPT // CH
        rem = RPT - nfull * CH
        for k in range(nfull):
            pltpu.sync_copy(rows0, acc_sh.at[pl.ds(s * RPT + k * CH, CH)])
        pltpu.sync_copy(rows0.at[pl.ds(0, rem)],
                        acc_sh.at[pl.ds(s * RPT + nfull * CH, rem)])
        if do_deg:
            for k in range(nfull):
                pltpu.sync_copy(zd_v, deg_sh.at[pl.ds(s * RPT + k * CH, CH)])
            pltpu.sync_copy(zd_v.at[pl.ds(0, rem)],
                            deg_sh.at[pl.ds(s * RPT + nfull * CH, rem)])
        pltpu.sync_copy(src_hbm.at[wid], src_v)
        pltpu.sync_copy(dst_hbm.at[wid], dst_v)
        plsc.subcore_barrier()

        def scat(buf, j):
            pltpu.sync_copy(buf, acc_sh.at[dst_v.at[j]], add=True)
            if do_deg:
                pltpu.sync_copy(ones_v, deg_sh.at[dst_v.at[j]], add=True)

        pltpu.async_copy(x_hbm.at[src_v.at[0]], rows0, sem0)

        def body(g, carry):
            j0 = 2 * g
            pltpu.async_copy(x_hbm.at[src_v.at[j0 + 1]], rows1, sem1)
            pltpu.make_async_copy(x_hbm.at[src_v.at[j0]], rows0, sem0).wait()
            scat(rows0, j0)
            pltpu.async_copy(x_hbm.at[src_v.at[j0 + 2]], rows0, sem0)
            pltpu.make_async_copy(x_hbm.at[src_v.at[j0 + 1]], rows1, sem1).wait()
            scat(rows1, j0 + 1)
            return carry

        lax.fori_loop(0, (CPW - 2) // 2, body, 0)
        pltpu.async_copy(x_hbm.at[src_v.at[CPW - 1]], rows1, sem1)
        pltpu.make_async_copy(x_hbm.at[src_v.at[CPW - 2]], rows0, sem0).wait()
        scat(rows0, CPW - 2)
        pltpu.make_async_copy(x_hbm.at[src_v.at[CPW - 1]], rows1, sem1).wait()
        scat(rows1, CPW - 1)
        plsc.subcore_barrier()

        ob = s * RPT
        pltpu.sync_copy(acc_sh.at[pl.ds(ob, RPT)], agg_out.at[c, pl.ds(ob, RPT)])
        if do_deg:
            pltpu.sync_copy(deg_sh.at[pl.ds(ob, RPT)], deg_out.at[c, pl.ds(ob, RPT)])

    return agg


_agg_with_deg = _make_agg(True)
_agg_no_deg = _make_agg(False)

_BN = 1000


def _self_body(x_ref, wr_ref, bl_ref, o_ref):
    o_ref[...] = (jnp.dot(x_ref[...], wr_ref[...],
                          preferred_element_type=jnp.float32) + bl_ref[...])


def _self(x, WrT, bl):
    return pl.pallas_call(
        _self_body,
        grid=(N // _BN,),
        in_specs=[
            pl.BlockSpec((_BN, D), lambda i: (i, 0)),
            pl.BlockSpec((D, D), lambda i: (0, 0)),
            pl.BlockSpec((1, D), lambda i: (0, 0)),
        ],
        out_specs=pl.BlockSpec((_BN, D), lambda i: (i, 0)),
        out_shape=jax.ShapeDtypeStruct((N, D), jnp.float32),
    )(x, WrT, bl)


def _comb_body(relu, agg_ref, deg_ref, s_ref, wl_ref, o_ref):
    agg = agg_ref[0] + agg_ref[1]
    deg = deg_ref[0, :, 0:1] + deg_ref[1, :, 0:1]
    mean = agg / jnp.maximum(deg, 1.0)
    y = jnp.dot(mean, wl_ref[...], preferred_element_type=jnp.float32) + s_ref[...]
    o_ref[...] = jnp.maximum(y, 0.0) if relu else y


def _comb(aggp, degp, s, WlT, relu):
    return pl.pallas_call(
        functools.partial(_comb_body, relu),
        grid=(N // _BN,),
        in_specs=[
            pl.BlockSpec((NC, _BN, D), lambda i: (0, i, 0)),
            pl.BlockSpec((NC, _BN, 16), lambda i: (0, i, 0)),
            pl.BlockSpec((_BN, D), lambda i: (i, 0)),
            pl.BlockSpec((D, D), lambda i: (0, 0)),
        ],
        out_specs=pl.BlockSpec((_BN, D), lambda i: (i, 0)),
        out_shape=jax.ShapeDtypeStruct((N, D), jnp.float32),
    )(aggp, degp, s, WlT)


def kernel(x, edge_index, Wl1, bl1, Wr1, Wl2, bl2, Wr2):
    src = edge_index[0]
    dst = edge_index[1]
    epw = E // NW
    pad_w = EPW - epw
    src_p = jnp.concatenate(
        [src.reshape(NW, epw), jnp.zeros((NW, pad_w), jnp.int32)], axis=1
    ).reshape(NW, CPW, CH)
    dummy = N + (jnp.arange(pad_w, dtype=jnp.int32) % (NACC - N))
    dst_p = jnp.concatenate(
        [dst.reshape(NW, epw), jnp.broadcast_to(dummy, (NW, pad_w))], axis=1
    ).reshape(NW, CPW, CH)
    s1 = _self(x, Wr1.T, bl1.reshape(1, D))
    agg1, deg = _agg_with_deg(x, src_p, dst_p)
    h1 = _comb(agg1, deg, s1, Wl1.T, relu=True)
    s2 = _self(h1, Wr2.T, bl2.reshape(1, D))
    (agg2,) = _agg_no_deg(h1, src_p, dst_p)
    out = _comb(agg2, deg, s2, Wl2.T, relu=False)
    return out


# speedup vs baseline: 1.1271x; 1.0224x over previous
import functools

import jax
import jax.numpy as jnp
from jax import lax
from jax.experimental import pallas as pl
from jax.experimental.pallas import tpu as pltpu
from jax.experimental.pallas import tpu_sc as plsc

N = 10000
D = 128
E = 320000

NC = 2
NS = 16
NW = NC * NS

CH = 64
CPW = 158
EPW = CPW * CH
EREAL = E // NW
NACC = 10016
RPT = NACC // NS


def _make_agg(do_deg):
    mesh = plsc.VectorSubcoreMesh(core_axis_name="c", subcore_axis_name="s")
    out_type = [jax.ShapeDtypeStruct((NC, NACC, D), jnp.float32)]
    scratch = [
        pltpu.VMEM((EPW,), jnp.int32),
        pltpu.VMEM((EPW,), jnp.int32),
        pltpu.VMEM((CH, D), jnp.float32),
        pltpu.VMEM((CH, D), jnp.float32),
        pltpu.VMEM_SHARED((NACC, D), jnp.float32),
        pltpu.SemaphoreType.DMA,
        pltpu.SemaphoreType.DMA,
    ]
    if do_deg:
        out_type.append(jax.ShapeDtypeStruct((NC, NACC, 16), jnp.float32))
        scratch += [
            pltpu.VMEM((CH, 16), jnp.float32),
            pltpu.VMEM((CH, 16), jnp.float32),
            pltpu.VMEM_SHARED((NACC, 16), jnp.float32),
        ]

    @functools.partial(
        pl.kernel, mesh=mesh, out_type=out_type, scratch_types=scratch,
        compiler_params=pltpu.CompilerParams(use_tc_tiling_on_sc=False))
    def agg(*refs):
        if do_deg:
            (x_hbm, ei_hbm,
             agg_out, deg_out, src_v, dst_v, rows0, rows1, acc_sh, sem0, sem1,
             ones_v, zd_v, deg_sh) = refs
        else:
            (x_hbm, ei_hbm,
             agg_out, src_v, dst_v, rows0, rows1, acc_sh, sem0, sem1) = refs
        c = lax.axis_index("c")
        s = lax.axis_index("s")
        wid = c * NS + s

        z16 = jnp.zeros((16,), jnp.float32)
        o16 = jnp.ones((16,), jnp.float32)

        def fill(i, carry):
            for j in range(D // 16):
                rows0[i, pl.ds(j * 16, 16)] = z16
            if do_deg:
                ones_v[i, :] = o16
                zd_v[i, :] = z16
            return carry

        lax.fori_loop(0, CH, fill, 0)

        z16i = jnp.zeros((16,), jnp.int32)
        dummy16 = N + jax.lax.iota(jnp.int32, 16)
        for k in range((EPW - EREAL) // 16):
            src_v[pl.ds(EREAL + k * 16, 16)] = z16i
            dst_v[pl.ds(EREAL + k * 16, 16)] = dummy16

        nfull = RPT // CH
        rem = RPT - nfull * CH
        for k in range(nfull):
            pltpu.sync_copy(rows0, acc_sh.at[pl.ds(s * RPT + k * CH, CH)])
        pltpu.sync_copy(rows0.at[pl.ds(0, rem)],
                        acc_sh.at[pl.ds(s * RPT + nfull * CH, rem)])
        if do_deg:
            for k in range(nfull):
                pltpu.sync_copy(zd_v, deg_sh.at[pl.ds(s * RPT + k * CH, CH)])
            pltpu.sync_copy(zd_v.at[pl.ds(0, rem)],
                            deg_sh.at[pl.ds(s * RPT + nfull * CH, rem)])
        pltpu.sync_copy(ei_hbm.at[0, pl.ds(wid * EREAL, EREAL)],
                        src_v.at[pl.ds(0, EREAL)])
        pltpu.sync_copy(ei_hbm.at[1, pl.ds(wid * EREAL, EREAL)],
                        dst_v.at[pl.ds(0, EREAL)])
        plsc.subcore_barrier()

        def idx(j):
            return pl.ds(j * CH, CH)

        def scat(buf, j):
            pltpu.sync_copy(buf, acc_sh.at[dst_v.at[idx(j)]], add=True)
            if do_deg:
                pltpu.sync_copy(ones_v, deg_sh.at[dst_v.at[idx(j)]], add=True)

        pltpu.async_copy(x_hbm.at[src_v.at[idx(0)]], rows0, sem0)

        def body(g, carry):
            j0 = 2 * g
            pltpu.async_copy(x_hbm.at[src_v.at[idx(j0 + 1)]], rows1, sem1)
            pltpu.make_async_copy(x_hbm.at[src_v.at[idx(j0)]], rows0, sem0).wait()
            scat(rows0, j0)
            pltpu.async_copy(x_hbm.at[src_v.at[idx(j0 + 2)]], rows0, sem0)
            pltpu.make_async_copy(x_hbm.at[src_v.at[idx(j0 + 1)]], rows1, sem1).wait()
            scat(rows1, j0 + 1)
            return carry

        lax.fori_loop(0, (CPW - 2) // 2, body, 0)
        pltpu.async_copy(x_hbm.at[src_v.at[idx(CPW - 1)]], rows1, sem1)
        pltpu.make_async_copy(x_hbm.at[src_v.at[idx(CPW - 2)]], rows0, sem0).wait()
        scat(rows0, CPW - 2)
        pltpu.make_async_copy(x_hbm.at[src_v.at[idx(CPW - 1)]], rows1, sem1).wait()
        scat(rows1, CPW - 1)
        plsc.subcore_barrier()

        ob = s * RPT
        pltpu.sync_copy(acc_sh.at[pl.ds(ob, RPT)], agg_out.at[c, pl.ds(ob, RPT)])
        if do_deg:
            pltpu.sync_copy(deg_sh.at[pl.ds(ob, RPT)], deg_out.at[c, pl.ds(ob, RPT)])

    return agg


_agg_with_deg = _make_agg(True)
_agg_no_deg = _make_agg(False)

_BN = 1000


def _self_body(x_ref, wr_ref, bl_ref, o_ref):
    o_ref[...] = (jnp.dot(x_ref[...], wr_ref[...],
                          preferred_element_type=jnp.float32) + bl_ref[...])


def _self(x, WrT, bl):
    return pl.pallas_call(
        _self_body,
        grid=(N // _BN,),
        in_specs=[
            pl.BlockSpec((_BN, D), lambda i: (i, 0)),
            pl.BlockSpec((D, D), lambda i: (0, 0)),
            pl.BlockSpec((1, D), lambda i: (0, 0)),
        ],
        out_specs=pl.BlockSpec((_BN, D), lambda i: (i, 0)),
        out_shape=jax.ShapeDtypeStruct((N, D), jnp.float32),
    )(x, WrT, bl)


def _comb_body(relu, agg_ref, deg_ref, s_ref, wl_ref, o_ref):
    agg = agg_ref[0] + agg_ref[1]
    deg = deg_ref[0, :, 0:1] + deg_ref[1, :, 0:1]
    mean = agg / jnp.maximum(deg, 1.0)
    y = jnp.dot(mean, wl_ref[...], preferred_element_type=jnp.float32) + s_ref[...]
    o_ref[...] = jnp.maximum(y, 0.0) if relu else y


def _comb(aggp, degp, s, WlT, relu):
    return pl.pallas_call(
        functools.partial(_comb_body, relu),
        grid=(N // _BN,),
        in_specs=[
            pl.BlockSpec((NC, _BN, D), lambda i: (0, i, 0)),
            pl.BlockSpec((NC, _BN, 16), lambda i: (0, i, 0)),
            pl.BlockSpec((_BN, D), lambda i: (i, 0)),
            pl.BlockSpec((D, D), lambda i: (0, 0)),
        ],
        out_specs=pl.BlockSpec((_BN, D), lambda i: (i, 0)),
        out_shape=jax.ShapeDtypeStruct((N, D), jnp.float32),
    )(aggp, degp, s, WlT)


def kernel(x, edge_index, Wl1, bl1, Wr1, Wl2, bl2, Wr2):
    s1 = _self(x, Wr1.T, bl1.reshape(1, D))
    agg1, deg = _agg_with_deg(x, edge_index)
    h1 = _comb(agg1, deg, s1, Wl1.T, relu=True)
    s2 = _self(h1, Wr2.T, bl2.reshape(1, D))
    (agg2,) = _agg_no_deg(h1, edge_index)
    out = _comb(agg2, deg, s2, Wl2.T, relu=False)
    return out
